# baseline (speedup 1.0000x reference)
import jax
import jax.numpy as jnp
from jax.experimental import pallas as pl

N = 10000
E = 160000
H = 32
L = 2


def _layernorm(x, g, b):
    mu = jnp.mean(x, axis=-1, keepdims=True)
    var = jnp.mean((x - mu) ** 2, axis=-1, keepdims=True)
    return (x - mu) / jnp.sqrt(var + 1e-5) * g + b


def _nnconv(nf, ew, src, dst, bias):
    w = ew.reshape(-1, H, H // 2)
    m = nf[src][:, :, None] * w
    agg = jax.ops.segment_max(m, dst, num_segments=N)
    agg = jnp.where(jnp.isfinite(agg), agg, 0.0)
    return jnp.sum(agg, axis=1) + bias


def _elu(x):
    return jnp.where(x > 0, x, jnp.exp(jnp.minimum(x, 0.0)) - 1.0)


def _mlp_kernel(gf_ref, w1_ref, b1_ref, w2_ref, b2_ref, o_ref):
    hid = _elu(gf_ref[...] @ w1_ref[...] + b1_ref[...])
    o_ref[...] = hid @ w2_ref[...] + b2_ref[...]


def kernel(node_inp, edge_inp, edge_index, graph_feat, Wn, bn, We, be, Wmi, bmi, Wmo, bmo, ln_g, ln_b, conv_bi, conv_bo, W1, b1, W2, b2):
    src = edge_index[0]
    dst = edge_index[1]
    h = jax.nn.elu(node_inp @ Wn + bn)
    e = jax.nn.elu(edge_inp @ We + be)
    for l in range(L):
        nf = _layernorm(h, ln_g[l], ln_b[l])
        ef = _layernorm(e, ln_g[l], ln_b[l])
        mi = _nnconv(nf, ef @ Wmi[l] + bmi[l], src, dst, conv_bi[l])
        mo = _nnconv(nf, ef @ Wmo[l] + bmo[l], dst, src, conv_bo[l])
        m = jnp.concatenate([mi, mo], axis=-1)
        h = jax.nn.elu(nf + m)
    readout = jnp.max(h, axis=0, keepdims=True)
    gf = jnp.concatenate([readout, graph_feat[None, :]], axis=-1)
    out = pl.pallas_call(
        _mlp_kernel,
        out_shape=jax.ShapeDtypeStruct((1, 1), jnp.float32),
    )(gf, W1, b1[None, :], W2, b2[None, :])
    return out


# speedup vs baseline: 5.5889x; 5.5889x over previous
import functools

import jax
import jax.numpy as jnp
from jax import lax
from jax.experimental import pallas as pl
from jax.experimental.pallas import tpu as pltpu
from jax.experimental.pallas import tpu_sc as plsc

N = 10000
E = 160000
H = 32
HO = H // 2
HH = H * HO

NHALF = N // 2
CH = 1600
NCHUNK = E // CH
BE = 1000
BN = 2000


def _elu(x):
    return jnp.where(x > 0, x, jnp.exp(jnp.minimum(x, 0.0)) - 1.0)


def _ln(x, g, b):
    mu = jnp.mean(x, axis=-1, keepdims=True)
    var = jnp.mean((x - mu) ** 2, axis=-1, keepdims=True)
    return (x - mu) / jnp.sqrt(var + 1e-5) * g + b


def _node0_body(x_ref, wn_ref, bn_ref, g_ref, b_ref, o_ref, ot_ref):
    h = _elu(jnp.dot(x_ref[...], wn_ref[...],
                     preferred_element_type=jnp.float32) + bn_ref[...])
    nf = _ln(h, g_ref[...], b_ref[...])
    o_ref[...] = nf
    ot_ref[...] = nf.T[:, None, :]


def _node0(node_inp, Wn, bn, g, b):
    return pl.pallas_call(
        _node0_body,
        out_shape=[jax.ShapeDtypeStruct((N, H), jnp.float32),
                   jax.ShapeDtypeStruct((H, 1, N), jnp.float32)],
    )(node_inp, Wn, bn[None], g[None], b[None])


def _transpose_body(x_ref, o_ref):
    o_ref[...] = x_ref[...].T[:, None, :]


def _transpose(nf):
    return pl.pallas_call(
        _transpose_body,
        out_shape=jax.ShapeDtypeStruct((H, 1, N), jnp.float32),
    )(nf)


def _edge_body(x_ref, we_ref, be_ref, g_ref, b_ref,
               wcat_ref, bcat_ref, ewi_ref, ewo_ref):
    e = _elu(x_ref[...] * we_ref[...] + be_ref[...])
    ef = _ln(e, g_ref[...], b_ref[...])
    ew = jnp.dot(ef, wcat_ref[...],
                 preferred_element_type=jnp.float32) + bcat_ref[...]
    ewi_ref[...] = (ew[:, :HH].reshape(BE, H, HO)
                    .swapaxes(0, 1).reshape(H, BE * HO))
    ewo_ref[...] = (ew[:, HH:].reshape(BE, H, HO)
                    .swapaxes(0, 1).reshape(H, BE * HO))


def _edge_stage(edge_inp, We, be, g, b, Wcat, bcat):
    full = lambda shape: pl.BlockSpec(shape, lambda i: (0,) * len(shape))
    return pl.pallas_call(
        _edge_body,
        grid=(E // BE,),
        in_specs=[
            pl.BlockSpec((BE, 1), lambda i: (i, 0)),
            full((1, H)), full((1, H)), full((1, H)), full((1, H)),
            full((H, 2 * HH)), full((1, 2 * HH)),
        ],
        out_specs=[pl.BlockSpec((H, BE * HO), lambda i: (0, i))] * 2,
        out_shape=[jax.ShapeDtypeStruct((H, E * HO), jnp.float32)] * 2,
    )(edge_inp, We, be[None], g[None], b[None], Wcat, bcat[None])


def _conv_body(ewi_hbm, ewo_hbm, src_hbm, dst_hbm, nft_hbm,
               aggi_hbm, aggo_hbm, table_v, acc, m_v, seg_v, gid_v):
    i = lax.axis_index("s") * 2 + lax.axis_index("c")
    neg_inf = jnp.full((HO,), -jnp.inf, dtype=jnp.float32)

    pltpu.sync_copy(nft_hbm.at[i, 0], table_v.at[pl.ds(0, N)])

    def run_conv(ew_h, gid_h, seg_h, out_h):
        for p in range(2):
            base = p * NHALF

            def init_body(r, c):
                acc[pl.ds(r * HO, HO)] = neg_inf
                return c
            lax.fori_loop(0, NHALF + 1, init_body, 0)

            def chunk_body(ci, c):
                e0 = ci * CH
                pltpu.sync_copy(seg_h.at[pl.ds(e0, CH)], seg_v)
                pltpu.sync_copy(gid_h.at[pl.ds(e0, CH)], gid_v)
                pltpu.sync_copy(ew_h.at[i, pl.ds(e0 * HO, CH * HO)], m_v)

                def group_body(g, c2):
                    k0 = g * HO
                    sv = seg_v[pl.ds(k0, HO)]
                    gv = gid_v[pl.ds(k0, HO)]
                    for j in range(HO):
                        d = sv[j] - base
                        ok = (d >= 0) & (d < NHALF)
                        rj = jnp.where(ok, d, jnp.int32(NHALF)) * HO
                        s = table_v[pl.ds(gv[j], HO)][0]
                        a = acc[pl.ds(rj, HO)]
                        acc[pl.ds(rj, HO)] = jnp.maximum(
                            a, s * m_v[pl.ds((k0 + j) * HO, HO)])
                    return c2
                lax.fori_loop(0, CH // HO, group_body, 0)
                return c
            lax.fori_loop(0, NCHUNK, chunk_body, 0)
            pltpu.sync_copy(acc.at[pl.ds(0, NHALF * HO)],
                            out_h.at[i, pl.ds(base * HO, NHALF * HO)])

    run_conv(ewi_hbm, src_hbm, dst_hbm, aggi_hbm)
    run_conv(ewo_hbm, dst_hbm, src_hbm, aggo_hbm)


_sc_conv = functools.partial(
    pl.kernel,
    mesh=plsc.VectorSubcoreMesh(core_axis_name="c", subcore_axis_name="s"),
    out_type=[jax.ShapeDtypeStruct((H, N * HO), jnp.float32),
              jax.ShapeDtypeStruct((H, N * HO), jnp.float32)],
    scratch_types=[
        pltpu.VMEM((N + HO,), jnp.float32),
        pltpu.VMEM(((NHALF + 1) * HO,), jnp.float32),
        pltpu.VMEM((CH * HO,), jnp.float32),
        pltpu.VMEM((CH,), jnp.int32),
        pltpu.VMEM((CH,), jnp.int32),
    ],
)(_conv_body)


def _node_body(ai_ref, ao_ref, nf_ref, cbi_ref, cbo_ref,
               g_ref, b_ref, oh_ref, onf_ref):
    ai = ai_ref[...].reshape(H, BN, HO)
    ao = ao_ref[...].reshape(H, BN, HO)
    ai = jnp.where(jnp.isfinite(ai), ai, 0.0)
    ao = jnp.where(jnp.isfinite(ao), ao, 0.0)
    mi = jnp.sum(ai, axis=0) + cbi_ref[...]
    mo = jnp.sum(ao, axis=0) + cbo_ref[...]
    h = _elu(nf_ref[...] + jnp.concatenate([mi, mo], axis=-1))
    oh_ref[...] = h
    onf_ref[...] = _ln(h, g_ref[...], b_ref[...])


def _node_stage(aggi, aggo, nf, cbi, cbo, g, b):
    full = lambda shape: pl.BlockSpec(shape, lambda i: (0,) * len(shape))
    return pl.pallas_call(
        _node_body,
        grid=(N // BN,),
        in_specs=[
            pl.BlockSpec((H, BN * HO), lambda i: (0, i)),
            pl.BlockSpec((H, BN * HO), lambda i: (0, i)),
            pl.BlockSpec((BN, H), lambda i: (i, 0)),
            full((1, HO)), full((1, HO)),
            full((1, H)), full((1, H)),
        ],
        out_specs=[pl.BlockSpec((BN, H), lambda i: (i, 0))] * 2,
        out_shape=[jax.ShapeDtypeStruct((N, H), jnp.float32)] * 2,
    )(aggi, aggo, nf, cbi[None], cbo[None], g[None], b[None])


def _readout_body(h_ref, gfeat_ref, w1_ref, b1_ref, w2_ref, b2_ref, o_ref):
    r = jnp.max(h_ref[...], axis=0, keepdims=True)
    gf = jnp.concatenate([r, gfeat_ref[...]], axis=-1)
    hid = _elu(jnp.dot(gf, w1_ref[...],
                       preferred_element_type=jnp.float32) + b1_ref[...])
    o_ref[...] = jnp.dot(hid, w2_ref[...],
                         preferred_element_type=jnp.float32) + b2_ref[...]


def _readout(h, graph_feat, W1, b1, W2, b2):
    return pl.pallas_call(
        _readout_body,
        out_shape=jax.ShapeDtypeStruct((1, 1), jnp.float32),
    )(h, graph_feat[None], W1, b1[None], W2, b2[None])


def kernel(node_inp, edge_inp, edge_index, graph_feat, Wn, bn, We, be,
           Wmi, bmi, Wmo, bmo, ln_g, ln_b, conv_bi, conv_bo, W1, b1, W2, b2):
    src = edge_index[0]
    dst = edge_index[1]

    Wcat = [jnp.concatenate([Wmi[l], Wmo[l]], axis=1) for l in range(2)]
    bcat = [jnp.concatenate([bmi[l], bmo[l]]) for l in range(2)]

    nf, nft = _node0(node_inp, Wn, bn, ln_g[0], ln_b[0])
    h = None
    for l in range(2):
        ewi, ewo = _edge_stage(edge_inp, We, be, ln_g[l], ln_b[l],
                               Wcat[l], bcat[l])
        aggi, aggo = _sc_conv(ewi, ewo, src, dst, nft)
        h, nf = _node_stage(aggi, aggo, nf, conv_bi[l], conv_bo[l],
                            ln_g[(l + 1) % 2], ln_b[(l + 1) % 2])
        if l == 0:
            nft = _transpose(nf)
    return _readout(h, graph_feat, W1, b1, W2, b2)
